# Initial kernel scaffold; baseline (speedup 1.0000x reference)
import jax
import jax.numpy as jnp
from jax.experimental import pallas as pl


def kernel(x, gate_w, w1, w3, w2):
    raise NotImplementedError("write your pallas kernel here")


# speedup vs baseline: 1.5650x; 1.5650x over previous
import functools

import jax
import jax.numpy as jnp
from jax.experimental import pallas as pl
from jax.experimental.pallas import tpu as pltpu

E = 16
TOP_K = 2
DIM = 1024
HIDDEN = 2048
BH = 512
HB = HIDDEN // BH


def _moe_body(x_ref, gate_w_ref, w1_ref, w3_ref, w2_ref, out_ref, coef_ref):
    e = pl.program_id(0)
    h = pl.program_id(1)

    @pl.when(jnp.logical_and(e == 0, h == 0))
    def _gating():
        xf = x_ref[...]
        logits = jax.lax.dot_general(
            xf, gate_w_ref[...], (((1,), (1,)), ((), ())),
            preferred_element_type=jnp.float32)
        m = jnp.max(logits, axis=-1, keepdims=True)
        ex = jnp.exp(logits - m)
        scores = ex / jnp.sum(ex, axis=-1, keepdims=True)
        idx = jax.lax.broadcasted_iota(jnp.int32, scores.shape, 1)
        m1 = jnp.max(scores, axis=-1, keepdims=True)
        i1 = jnp.min(jnp.where(scores == m1, idx, E), axis=-1, keepdims=True)
        masked = jnp.where(idx == i1, -1.0, scores)
        m2 = jnp.max(masked, axis=-1, keepdims=True)
        i2 = jnp.min(jnp.where(masked == m2, idx, E), axis=-1, keepdims=True)
        e2 = jnp.exp(m2 - m1)
        denom = 1.0 + e2
        wa = 1.0 / denom
        wb = e2 / denom
        coef_ref[...] = jnp.where(idx == i1, wa, 0.0) + jnp.where(idx == i2, wb, 0.0)
        out_ref[...] = jnp.zeros_like(out_ref)

    xf = x_ref[...]
    t1 = jax.lax.dot_general(
        xf, w1_ref[0], (((1,), (1,)), ((), ())),
        preferred_element_type=jnp.float32)
    t3 = jax.lax.dot_general(
        xf, w3_ref[0], (((1,), (1,)), ((), ())),
        preferred_element_type=jnp.float32)
    act = t1 * jax.lax.logistic(t1) * t3
    coef = coef_ref[...]
    eidx = jax.lax.broadcasted_iota(jnp.int32, coef.shape, 1)
    ce = jnp.sum(jnp.where(eidx == e, coef, 0.0), axis=1, keepdims=True)
    act = act * ce
    out_ref[...] += jax.lax.dot_general(
        act, w2_ref[0], (((1,), (1,)), ((), ())),
        preferred_element_type=jnp.float32)


@jax.jit
def _moe(xf, gate_w, w1, w3, w2):
    return pl.pallas_call(
        _moe_body,
        grid=(E, HB),
        in_specs=[
            pl.BlockSpec((64, DIM), lambda e, h: (0, 0)),
            pl.BlockSpec((E, DIM), lambda e, h: (0, 0)),
            pl.BlockSpec((1, BH, DIM), lambda e, h: (e, h, 0)),
            pl.BlockSpec((1, BH, DIM), lambda e, h: (e, h, 0)),
            pl.BlockSpec((1, DIM, BH), lambda e, h: (e, 0, h)),
        ],
        out_specs=pl.BlockSpec((64, DIM), lambda e, h: (0, 0)),
        out_shape=jax.ShapeDtypeStruct((64, DIM), jnp.float32),
        scratch_shapes=[pltpu.VMEM((64, E), jnp.float32)],
    )(xf, gate_w, w1, w3, w2)


def kernel(x, gate_w, w1, w3, w2):
    orig_shape = x.shape
    xf = x.reshape(-1, x.shape[-1])
    return _moe(xf, gate_w, w1, w3, w2).reshape(orig_shape)


# speedup vs baseline: 1.7492x; 1.1177x over previous
import functools

import jax
import jax.numpy as jnp
from jax.experimental import pallas as pl
from jax.experimental.pallas import tpu as pltpu

E = 16
TOP_K = 2
DIM = 1024
HIDDEN = 2048
BH = 1024
HB = HIDDEN // BH


def _moe_body(x_ref, gate_w_ref, w1_ref, w3_ref, w2_ref, out_ref, coef_ref):
    e = pl.program_id(0)
    h = pl.program_id(1)

    @pl.when(jnp.logical_and(e == 0, h == 0))
    def _gating():
        xf = x_ref[...]
        logits = jax.lax.dot_general(
            xf, gate_w_ref[...], (((1,), (1,)), ((), ())),
            preferred_element_type=jnp.float32)
        m = jnp.max(logits, axis=-1, keepdims=True)
        ex = jnp.exp(logits - m)
        scores = ex / jnp.sum(ex, axis=-1, keepdims=True)
        idx = jax.lax.broadcasted_iota(jnp.int32, scores.shape, 1)
        m1 = jnp.max(scores, axis=-1, keepdims=True)
        i1 = jnp.min(jnp.where(scores == m1, idx, E), axis=-1, keepdims=True)
        masked = jnp.where(idx == i1, -1.0, scores)
        m2 = jnp.max(masked, axis=-1, keepdims=True)
        i2 = jnp.min(jnp.where(masked == m2, idx, E), axis=-1, keepdims=True)
        e2 = jnp.exp(m2 - m1)
        denom = 1.0 + e2
        wa = 1.0 / denom
        wb = e2 / denom
        coef_ref[...] = jnp.where(idx == i1, wa, 0.0) + jnp.where(idx == i2, wb, 0.0)
        out_ref[...] = jnp.zeros_like(out_ref)

    xf = x_ref[...]
    t1 = jax.lax.dot_general(
        xf, w1_ref[0], (((1,), (1,)), ((), ())),
        preferred_element_type=jnp.float32)
    t3 = jax.lax.dot_general(
        xf, w3_ref[0], (((1,), (1,)), ((), ())),
        preferred_element_type=jnp.float32)
    act = t1 * jax.lax.logistic(t1) * t3
    coef = coef_ref[...]
    eidx = jax.lax.broadcasted_iota(jnp.int32, coef.shape, 1)
    ce = jnp.sum(jnp.where(eidx == e, coef, 0.0), axis=1, keepdims=True)
    act = act * ce
    out_ref[...] += jax.lax.dot_general(
        act, w2_ref[0], (((1,), (1,)), ((), ())),
        preferred_element_type=jnp.float32)


@jax.jit
def _moe(xf, gate_w, w1, w3, w2):
    return pl.pallas_call(
        _moe_body,
        grid=(E, HB),
        in_specs=[
            pl.BlockSpec((64, DIM), lambda e, h: (0, 0)),
            pl.BlockSpec((E, DIM), lambda e, h: (0, 0)),
            pl.BlockSpec((1, BH, DIM), lambda e, h: (e, h, 0)),
            pl.BlockSpec((1, BH, DIM), lambda e, h: (e, h, 0)),
            pl.BlockSpec((1, DIM, BH), lambda e, h: (e, 0, h)),
        ],
        out_specs=pl.BlockSpec((64, DIM), lambda e, h: (0, 0)),
        out_shape=jax.ShapeDtypeStruct((64, DIM), jnp.float32),
        scratch_shapes=[pltpu.VMEM((64, E), jnp.float32)],
    )(xf, gate_w, w1, w3, w2)


def kernel(x, gate_w, w1, w3, w2):
    orig_shape = x.shape
    xf = x.reshape(-1, x.shape[-1])
    return _moe(xf, gate_w, w1, w3, w2).reshape(orig_shape)
